# baseline (speedup 1.0000x reference)
import jax
import jax.numpy as jnp
from jax.experimental import pallas as pl
from jax.experimental.pallas import tpu as pltpu

_TN = 256


def _chamfer_body(x_ref, yt_ref, loss_ref, idmin_ref, lsum_ref):
    b = pl.program_id(0)
    i = pl.program_id(1)
    nb = pl.num_programs(0)
    ni = pl.num_programs(1)

    x = x_ref[0]
    yt = yt_ref[0]

    x2 = (x[:, 0:1] * x[:, 0:1] + x[:, 1:2] * x[:, 1:2]
          + x[:, 2:3] * x[:, 2:3])
    y2 = (yt[0:1, :] * yt[0:1, :] + yt[1:2, :] * yt[1:2, :]
          + yt[2:3, :] * yt[2:3, :])

    xb = x.astype(jnp.bfloat16).astype(jnp.float32)
    yb = yt.astype(jnp.bfloat16).astype(jnp.float32)
    xy = (xb[:, 0:1] * yb[0:1, :] + xb[:, 1:2] * yb[1:2, :]
          + xb[:, 2:3] * yb[2:3, :])

    dist = jnp.maximum(x2 + y2 - 2.0 * xy, 0.0)

    row_min = jnp.min(dist, axis=1)
    col_min = jnp.min(dist, axis=0, keepdims=True)

    @pl.when(jnp.logical_and(b == 0, i == 0))
    def _():
        lsum_ref[0, 0] = 0.0

    @pl.when(i == 0)
    def _():
        idmin_ref[...] = col_min

    @pl.when(i != 0)
    def _():
        idmin_ref[...] = jnp.minimum(idmin_ref[...], col_min)

    lsum_ref[0, 0] += jnp.sum(row_min)

    @pl.when(i == ni - 1)
    def _():
        lsum_ref[0, 0] += jnp.sum(idmin_ref[...])

    @pl.when(jnp.logical_and(b == nb - 1, i == ni - 1))
    def _():
        loss_ref[0, 0] = lsum_ref[0, 0]


def kernel(x, y):
    B, N, _ = x.shape
    M = y.shape[1]
    yt = jnp.swapaxes(y, 1, 2)

    loss = pl.pallas_call(
        _chamfer_body,
        grid=(B, N // _TN),
        in_specs=[
            pl.BlockSpec((1, _TN, 3), lambda b, i: (b, i, 0)),
            pl.BlockSpec((1, 3, M), lambda b, i: (b, 0, 0)),
        ],
        out_specs=pl.BlockSpec(
            (1, 1), lambda b, i: (0, 0), memory_space=pltpu.SMEM),
        out_shape=jax.ShapeDtypeStruct((1, 1), jnp.float32),
        scratch_shapes=[
            pltpu.VMEM((1, M), jnp.float32),
            pltpu.SMEM((1, 1), jnp.float32),
        ],
    )(x, yt)
    return loss[0, 0] / x.shape[0]


# speedup vs baseline: 1.8723x; 1.8723x over previous
import jax
import jax.numpy as jnp
from jax.experimental import pallas as pl
from jax.experimental.pallas import tpu as pltpu

_TN = 512
_K = 16


def _bf16_split3(v):
    hi = v.astype(jnp.bfloat16)
    r = v - hi.astype(jnp.float32)
    mid = r.astype(jnp.bfloat16)
    lo = (r - mid.astype(jnp.float32)).astype(jnp.bfloat16)
    return hi, mid, lo


def _chamfer_body(xa_ref, ya_ref, loss_ref, idmin_ref, lsum_ref):
    b = pl.program_id(0)
    i = pl.program_id(1)
    nb = pl.num_programs(0)
    ni = pl.num_programs(1)

    xa = xa_ref[0]
    ya = ya_ref[0]

    dist = jax.lax.dot_general(
        xa, ya, (((1,), (0,)), ((), ())),
        preferred_element_type=jnp.float32)

    row_min = jnp.maximum(jnp.min(dist, axis=1), 0.0)
    col_min = jnp.maximum(jnp.min(dist, axis=0, keepdims=True), 0.0)

    @pl.when(jnp.logical_and(b == 0, i == 0))
    def _():
        lsum_ref[0, 0] = 0.0

    @pl.when(i == 0)
    def _():
        idmin_ref[...] = col_min

    @pl.when(i != 0)
    def _():
        idmin_ref[...] = jnp.minimum(idmin_ref[...], col_min)

    lsum_ref[0, 0] += jnp.sum(row_min)

    @pl.when(i == ni - 1)
    def _():
        lsum_ref[0, 0] += jnp.sum(idmin_ref[...])

    @pl.when(jnp.logical_and(b == nb - 1, i == ni - 1))
    def _():
        loss_ref[0, 0] = lsum_ref[0, 0]


def kernel(x, y):
    B, N, _ = x.shape
    M = y.shape[1]
    f32 = jnp.float32

    xb = x.astype(jnp.bfloat16)
    yb = y.astype(jnp.bfloat16)
    x2 = jnp.sum(x * x, axis=-1)
    y2 = jnp.sum(y * y, axis=-1)
    xh, xm, xl = _bf16_split3(x2)
    yh, ym, yl = _bf16_split3(y2)
    onesx = jnp.ones((B, N), jnp.bfloat16)
    onesy = jnp.ones((B, M), jnp.bfloat16)
    zerosx = jnp.zeros((B, N, _K - 9), jnp.bfloat16)
    zerosy = jnp.zeros((B, M, _K - 9), jnp.bfloat16)

    xa = jnp.concatenate(
        [(-2.0 * xb.astype(f32)).astype(jnp.bfloat16),
         xh[..., None], xm[..., None], xl[..., None],
         onesx[..., None], onesx[..., None], onesx[..., None],
         zerosx], axis=-1)
    ya = jnp.concatenate(
        [yb,
         onesy[..., None], onesy[..., None], onesy[..., None],
         yh[..., None], ym[..., None], yl[..., None],
         zerosy], axis=-1)
    yat = jnp.swapaxes(ya, 1, 2)

    loss = pl.pallas_call(
        _chamfer_body,
        grid=(B, N // _TN),
        in_specs=[
            pl.BlockSpec((1, _TN, _K), lambda b, i: (b, i, 0)),
            pl.BlockSpec((1, _K, M), lambda b, i: (b, 0, 0)),
        ],
        out_specs=pl.BlockSpec(
            (1, 1), lambda b, i: (0, 0), memory_space=pltpu.SMEM),
        out_shape=jax.ShapeDtypeStruct((1, 1), jnp.float32),
        scratch_shapes=[
            pltpu.VMEM((1, M), jnp.float32),
            pltpu.SMEM((1, 1), jnp.float32),
        ],
    )(xa, yat)
    return loss[0, 0] / x.shape[0]
